# baseline (speedup 1.0000x reference)
import functools

import jax
import jax.numpy as jnp
from jax import lax
from jax.experimental import pallas as pl
from jax.experimental.pallas import tpu as pltpu
from jax.experimental.pallas import tpu_sc as plsc

_NC = 2
_NS = 16
_NW = _NC * _NS

_USER_DIM = 64
_GENDER_DIM = 8
_AGE_DIM = 16
_OCC_DIM = 32
_FINAL = 128


def _sc_gather(sample, map_gender, map_age, map_occupation,
               user_id_emb, gender_pad, age_pad, occ_pad):
    batch = sample.shape[0]
    bpw = batch // _NW

    mesh = plsc.VectorSubcoreMesh(core_axis_name="c", subcore_axis_name="s")
    out_type = (
        jax.ShapeDtypeStruct((batch, _USER_DIM), jnp.float32),
        jax.ShapeDtypeStruct((batch, _FINAL), jnp.float32),
        jax.ShapeDtypeStruct((batch, _FINAL), jnp.float32),
        jax.ShapeDtypeStruct((batch, _FINAL), jnp.float32),
    )

    @functools.partial(
        pl.kernel,
        out_type=out_type,
        mesh=mesh,
        scratch_types=[
            pltpu.VMEM((bpw,), jnp.int32),
            pltpu.VMEM((bpw,), jnp.int32),
            pltpu.VMEM((bpw,), jnp.int32),
            pltpu.VMEM((bpw,), jnp.int32),
            pltpu.VMEM((bpw, _USER_DIM), jnp.float32),
            pltpu.VMEM((bpw, _FINAL), jnp.float32),
            pltpu.VMEM((bpw, _FINAL), jnp.float32),
            pltpu.VMEM((bpw, _FINAL), jnp.float32),
            pltpu.SemaphoreType.DMA,
            pltpu.SemaphoreType.DMA,
            pltpu.SemaphoreType.DMA,
            pltpu.SemaphoreType.DMA,
        ],
    )
    def gather_kernel(sample_h, mg_h, ma_h, mo_h, ue_h, ge_h, ae_h, oe_h,
                      fu_o, fg_o, fa_o, fo_o,
                      idx_v, gid_v, aid_v, oid_v, fu_v, fg_v, fa_v, fo_v,
                      sem_u, sem_g, sem_a, sem_o):
        wid = lax.axis_index("s") * _NC + lax.axis_index("c")
        base = wid * bpw
        pltpu.sync_copy(sample_h.at[pl.ds(base, bpw)], idx_v)
        cp_g = pltpu.async_copy(mg_h.at[idx_v], gid_v, sem_g)
        cp_a = pltpu.async_copy(ma_h.at[idx_v], aid_v, sem_a)
        cp_o = pltpu.async_copy(mo_h.at[idx_v], oid_v, sem_o)

        def ugroup(g, carry):
            idx16 = idx_v[pl.ds(g * 16, 16)]
            for j in range(16):
                r = idx16[j]
                pltpu.async_copy(ue_h.at[pl.ds(r, 1), :],
                                 fu_v.at[pl.ds(g * 16 + j, 1), :], sem_u)

            @pl.when(g >= 1)
            def _():
                for j in range(16):
                    pltpu.make_async_copy(
                        ue_h.at[pl.ds(0, 1), :],
                        fu_v.at[pl.ds((g - 1) * 16 + j, 1), :],
                        sem_u).wait()
            return carry
        lax.fori_loop(0, bpw // 16, ugroup, 0)

        cp_g.wait()
        cp_g2 = pltpu.async_copy(ge_h.at[gid_v], fg_v, sem_g)
        cp_a.wait()
        cp_a2 = pltpu.async_copy(ae_h.at[aid_v], fa_v, sem_a)
        cp_o.wait()
        cp_o2 = pltpu.async_copy(oe_h.at[oid_v], fo_v, sem_o)

        for j in range(16):
            pltpu.make_async_copy(ue_h.at[pl.ds(0, 1), :],
                                  fu_v.at[pl.ds(bpw - 16 + j, 1), :],
                                  sem_u).wait()
        pltpu.sync_copy(fu_v, fu_o.at[pl.ds(base, bpw)])
        cp_g2.wait()
        pltpu.sync_copy(fg_v, fg_o.at[pl.ds(base, bpw)])
        cp_a2.wait()
        pltpu.sync_copy(fa_v, fa_o.at[pl.ds(base, bpw)])
        cp_o2.wait()
        pltpu.sync_copy(fo_v, fo_o.at[pl.ds(base, bpw)])

    return gather_kernel(sample, map_gender, map_age, map_occupation,
                         user_id_emb, gender_pad, age_pad, occ_pad)


def _tc_mlp(fu, fg, fa, fo, W, b):
    batch = fu.shape[0]
    bm = 1024

    def body(fu_r, fg_r, fa_r, fo_r, w_r, b_r, o_r):
        w = w_r[...]
        acc = jnp.dot(fu_r[...], w[0:64], preferred_element_type=jnp.float32)
        acc += jnp.dot(fg_r[...][:, :_GENDER_DIM], w[64:72],
                       preferred_element_type=jnp.float32)
        acc += jnp.dot(fa_r[...][:, :_AGE_DIM], w[72:88],
                       preferred_element_type=jnp.float32)
        acc += jnp.dot(fo_r[...][:, :_OCC_DIM], w[88:120],
                       preferred_element_type=jnp.float32)
        o_r[...] = jnp.maximum(acc + b_r[...].reshape(1, _FINAL), 0.0)

    return pl.pallas_call(
        body,
        grid=(batch // bm,),
        in_specs=[
            pl.BlockSpec((bm, _USER_DIM), lambda i: (i, 0)),
            pl.BlockSpec((bm, _FINAL), lambda i: (i, 0)),
            pl.BlockSpec((bm, _FINAL), lambda i: (i, 0)),
            pl.BlockSpec((bm, _FINAL), lambda i: (i, 0)),
            pl.BlockSpec((120, _FINAL), lambda i: (0, 0)),
            pl.BlockSpec((_FINAL,), lambda i: (0,)),
        ],
        out_specs=pl.BlockSpec((bm, _FINAL), lambda i: (i, 0)),
        out_shape=jax.ShapeDtypeStruct((batch, _FINAL), jnp.float32),
    )(fu, fg, fa, fo, W, b)


def kernel(sample, map_gender, map_age, map_occupation, user_id_emb,
           gender_emb, age_emb, occupation_emb, W, b):
    gender_pad = jnp.pad(gender_emb, ((0, 0), (0, _FINAL - _GENDER_DIM)))
    age_pad = jnp.pad(age_emb, ((0, 0), (0, _FINAL - _AGE_DIM)))
    occ_pad = jnp.pad(occupation_emb, ((0, 0), (0, _FINAL - _OCC_DIM)))
    fu, fg, fa, fo = _sc_gather(sample, map_gender, map_age, map_occupation,
                                user_id_emb, gender_pad, age_pad, occ_pad)
    return _tc_mlp(fu, fg, fa, fo, W, b)


# speedup vs baseline: 1.0059x; 1.0059x over previous
import functools

import jax
import jax.numpy as jnp
from jax import lax
from jax.experimental import pallas as pl
from jax.experimental.pallas import tpu as pltpu
from jax.experimental.pallas import tpu_sc as plsc

_NC = 2
_NS = 16
_NW = _NC * _NS

_USER_DIM = 64
_GENDER_DIM = 8
_AGE_DIM = 16
_OCC_DIM = 32
_FINAL = 128


def _sc_gather(sample, map_gender, map_age, map_occupation,
               user_id_emb, gender_pad, age_pad, occ_pad):
    batch = sample.shape[0]
    bpw = batch // _NW

    mesh = plsc.VectorSubcoreMesh(core_axis_name="c", subcore_axis_name="s")
    out_type = (
        jax.ShapeDtypeStruct((batch, _USER_DIM), jnp.float32),
        jax.ShapeDtypeStruct((batch, _FINAL), jnp.float32),
        jax.ShapeDtypeStruct((batch, _FINAL), jnp.float32),
        jax.ShapeDtypeStruct((batch, _FINAL), jnp.float32),
    )

    @functools.partial(
        pl.kernel,
        out_type=out_type,
        mesh=mesh,
        scratch_types=[
            pltpu.VMEM((bpw,), jnp.int32),
            pltpu.VMEM((bpw,), jnp.int32),
            pltpu.VMEM((bpw,), jnp.int32),
            pltpu.VMEM((bpw,), jnp.int32),
            pltpu.VMEM((bpw, _USER_DIM), jnp.float32),
            pltpu.VMEM((bpw, _FINAL), jnp.float32),
            pltpu.VMEM((bpw, _FINAL), jnp.float32),
            pltpu.VMEM((bpw, _FINAL), jnp.float32),
            pltpu.SemaphoreType.DMA,
            pltpu.SemaphoreType.DMA,
            pltpu.SemaphoreType.DMA,
            pltpu.SemaphoreType.DMA,
        ],
    )
    def gather_kernel(sample_h, mg_h, ma_h, mo_h, ue_h, ge_h, ae_h, oe_h,
                      fu_o, fg_o, fa_o, fo_o,
                      idx_v, gid_v, aid_v, oid_v, fu_v, fg_v, fa_v, fo_v,
                      sem_u, sem_g, sem_a, sem_o):
        wid = lax.axis_index("s") * _NC + lax.axis_index("c")
        base = wid * bpw
        pltpu.sync_copy(sample_h.at[pl.ds(base, bpw)], idx_v)
        cp_g = pltpu.async_copy(mg_h.at[idx_v], gid_v, sem_g)
        cp_a = pltpu.async_copy(ma_h.at[idx_v], aid_v, sem_a)
        cp_o = pltpu.async_copy(mo_h.at[idx_v], oid_v, sem_o)

        def ugroup(g, carry):
            idx16 = idx_v[pl.ds(g * 16, 16)]
            for j in range(16):
                r = idx16[j]
                pltpu.async_copy(ue_h.at[pl.ds(r, 1), :],
                                 fu_v.at[pl.ds(g * 16 + j, 1), :], sem_u)

            @pl.when(g >= 1)
            def _():
                pltpu.make_async_copy(
                    ue_h.at[pl.ds(0, 16), :],
                    fu_v.at[pl.ds((g - 1) * 16, 16), :], sem_u).wait()
            return carry
        lax.fori_loop(0, bpw // 16, ugroup, 0)

        cp_g.wait()
        cp_g2 = pltpu.async_copy(ge_h.at[gid_v], fg_v, sem_g)
        cp_a.wait()
        cp_a2 = pltpu.async_copy(ae_h.at[aid_v], fa_v, sem_a)
        cp_o.wait()
        cp_o2 = pltpu.async_copy(oe_h.at[oid_v], fo_v, sem_o)

        pltpu.make_async_copy(ue_h.at[pl.ds(0, 16), :],
                              fu_v.at[pl.ds(bpw - 16, 16), :], sem_u).wait()
        pltpu.sync_copy(fu_v, fu_o.at[pl.ds(base, bpw)])
        cp_g2.wait()
        pltpu.sync_copy(fg_v, fg_o.at[pl.ds(base, bpw)])
        cp_a2.wait()
        pltpu.sync_copy(fa_v, fa_o.at[pl.ds(base, bpw)])
        cp_o2.wait()
        pltpu.sync_copy(fo_v, fo_o.at[pl.ds(base, bpw)])

    return gather_kernel(sample, map_gender, map_age, map_occupation,
                         user_id_emb, gender_pad, age_pad, occ_pad)


def _tc_mlp(fu, fg, fa, fo, W, b):
    batch = fu.shape[0]
    bm = 1024

    def body(fu_r, fg_r, fa_r, fo_r, w_r, b_r, o_r):
        w = w_r[...]
        acc = jnp.dot(fu_r[...], w[0:64], preferred_element_type=jnp.float32)
        acc += jnp.dot(fg_r[...][:, :_GENDER_DIM], w[64:72],
                       preferred_element_type=jnp.float32)
        acc += jnp.dot(fa_r[...][:, :_AGE_DIM], w[72:88],
                       preferred_element_type=jnp.float32)
        acc += jnp.dot(fo_r[...][:, :_OCC_DIM], w[88:120],
                       preferred_element_type=jnp.float32)
        o_r[...] = jnp.maximum(acc + b_r[...].reshape(1, _FINAL), 0.0)

    return pl.pallas_call(
        body,
        grid=(batch // bm,),
        in_specs=[
            pl.BlockSpec((bm, _USER_DIM), lambda i: (i, 0)),
            pl.BlockSpec((bm, _FINAL), lambda i: (i, 0)),
            pl.BlockSpec((bm, _FINAL), lambda i: (i, 0)),
            pl.BlockSpec((bm, _FINAL), lambda i: (i, 0)),
            pl.BlockSpec((120, _FINAL), lambda i: (0, 0)),
            pl.BlockSpec((_FINAL,), lambda i: (0,)),
        ],
        out_specs=pl.BlockSpec((bm, _FINAL), lambda i: (i, 0)),
        out_shape=jax.ShapeDtypeStruct((batch, _FINAL), jnp.float32),
    )(fu, fg, fa, fo, W, b)


def kernel(sample, map_gender, map_age, map_occupation, user_id_emb,
           gender_emb, age_emb, occupation_emb, W, b):
    gender_pad = jnp.pad(gender_emb, ((0, 0), (0, _FINAL - _GENDER_DIM)))
    age_pad = jnp.pad(age_emb, ((0, 0), (0, _FINAL - _AGE_DIM)))
    occ_pad = jnp.pad(occupation_emb, ((0, 0), (0, _FINAL - _OCC_DIM)))
    fu, fg, fa, fo = _sc_gather(sample, map_gender, map_age, map_occupation,
                                user_id_emb, gender_pad, age_pad, occ_pad)
    return _tc_mlp(fu, fg, fa, fo, W, b)
